# baseline (speedup 1.0000x reference)
import functools

import jax
import jax.numpy as jnp
from jax import lax
from jax.experimental import pallas as pl
from jax.experimental.pallas import tpu as pltpu
from jax.experimental.pallas import tpu_sc as plsc

_VOCAB = 100000
_EMBED = 128
_BATCH = 4096
_HIST = 200
_B = _BATCH * _HIST
_NC = 2
_NS = 16
_NW = _NC * _NS
_B_PER_W = _B // _NW
_CHUNK = 128
_NCHUNK = _B_PER_W // _CHUNK
_NB = 4
_PACK = _EMBED // 2

_mesh = plsc.VectorSubcoreMesh(core_axis_name="c", subcore_axis_name="s")


@functools.partial(
    pl.kernel,
    out_type=jax.ShapeDtypeStruct((_B, _EMBED), jnp.float32),
    mesh=_mesh,
    compiler_params=pltpu.CompilerParams(use_tc_tiling_on_sc=False),
    scratch_types=[
        pltpu.VMEM((_B_PER_W,), jnp.int32),
        [pltpu.VMEM((_CHUNK, _PACK), jnp.int32) for _ in range(_NB)],
        [pltpu.VMEM((_CHUNK, _EMBED), jnp.float32) for _ in range(_NB)],
        [pltpu.SemaphoreType.DMA for _ in range(_NB)],
        [pltpu.SemaphoreType.DMA for _ in range(_NB)],
    ],
)
def _gather_kernel(idx_hbm, packed_hbm, out_hbm, idx_v, gbufs, obufs,
                   gsems, ssems):
    wid = lax.axis_index("s") * _NC + lax.axis_index("c")
    base = wid * _B_PER_W

    pltpu.sync_copy(idx_hbm.at[pl.ds(base, _B_PER_W)], idx_v)

    def gather(c, b):
        pltpu.async_copy(
            packed_hbm.at[idx_v.at[pl.ds(c * _CHUNK, _CHUNK)]], gbufs[b],
            gsems[b])

    def wait_gather(b):
        pltpu.make_async_copy(gbufs[b], out_hbm.at[pl.ds(0, _CHUNK // 2)],
                              gsems[b]).wait()

    def wait_store(b):
        pltpu.make_async_copy(obufs[b], out_hbm.at[pl.ds(0, _CHUNK)],
                              ssems[b]).wait()

    def convert(b):
        gbuf, obuf = gbufs[b], obufs[b]

        def rows(i, carry):
            r0 = i * 8
            for dr in range(8):
                r = r0 + dr
                for v in range(_PACK // 16):
                    x = gbuf[r, pl.ds(16 * v, 16)]
                    obuf[r, pl.ds(32 * v, 16)] = lax.bitcast_convert_type(
                        jnp.left_shift(x, 16), jnp.float32)
                    obuf[r, pl.ds(32 * v + 16, 16)] = lax.bitcast_convert_type(
                        jnp.bitwise_and(x, jnp.int32(-65536)), jnp.float32)
            return carry

        lax.fori_loop(0, _CHUNK // 8, rows, 0)

    def step(c, b, first_ring):
        wait_gather(b)
        if not first_ring:
            wait_store(b)
        convert(b)
        pltpu.async_copy(obufs[b],
                         out_hbm.at[pl.ds(base + c * _CHUNK, _CHUNK)],
                         ssems[b])
        ng = c + _NB
        if isinstance(c, int):
            if ng < _NCHUNK:
                gather(ng, b)
        else:
            @pl.when(ng < _NCHUNK)
            def _():
                gather(ng, b)

    for b in range(_NB):
        gather(b, b)

    for b in range(_NB):
        step(b, b, first_ring=True)

    def body(r, carry):
        for b in range(_NB):
            step(r * _NB + b, b, False)
        return carry

    lax.fori_loop(1, _NCHUNK // _NB, body, 0)

    for b in range(_NB):
        wait_store(b)


def kernel(indices, table):
    flat_idx = indices.reshape(_B).astype(jnp.int32)
    tb = table.astype(jnp.bfloat16).reshape(_VOCAB, _EMBED // 32, 2, 16)
    lo = lax.bitcast_convert_type(tb[:, :, 0, :], jnp.uint16)
    hi = lax.bitcast_convert_type(tb[:, :, 1, :], jnp.uint16)
    words = (hi.astype(jnp.uint32) << 16) | lo.astype(jnp.uint32)
    packed = lax.bitcast_convert_type(words, jnp.int32).reshape(_VOCAB, _PACK)
    out = _gather_kernel(flat_idx, packed)
    return out.reshape(_BATCH, _HIST, _EMBED)


# speedup vs baseline: 1.8955x; 1.8955x over previous
import functools

import jax
import jax.numpy as jnp
from jax import lax
from jax.experimental import pallas as pl
from jax.experimental.pallas import tpu as pltpu
from jax.experimental.pallas import tpu_sc as plsc

_VOCAB = 100000
_EMBED = 128
_BATCH = 4096
_HIST = 200
_B = _BATCH * _HIST
_NC = 2
_NS = 16
_NW = _NC * _NS
_B_PER_W = _B // _NW
_CHUNK = 160
_NCHUNK = _B_PER_W // _CHUNK
_NB = 5

_mesh = plsc.VectorSubcoreMesh(core_axis_name="c", subcore_axis_name="s")


@functools.partial(
    pl.kernel,
    out_type=jax.ShapeDtypeStruct((_B, _EMBED), jnp.float32),
    mesh=_mesh,
    scratch_types=[
        pltpu.VMEM((_B_PER_W,), jnp.int32),
        [pltpu.VMEM((_CHUNK, _EMBED), jnp.float32) for _ in range(_NB)],
        [pltpu.SemaphoreType.DMA for _ in range(_NB)],
        [pltpu.SemaphoreType.DMA for _ in range(_NB)],
    ],
)
def _gather_kernel(idx_hbm, table_hbm, out_hbm, idx_v, bufs, gsems, ssems):
    wid = lax.axis_index("s") * _NC + lax.axis_index("c")
    base = wid * _B_PER_W

    pltpu.sync_copy(idx_hbm.at[pl.ds(base, _B_PER_W)], idx_v)

    def gather(c, buf, sem):
        pltpu.async_copy(
            table_hbm.at[idx_v.at[pl.ds(c * _CHUNK, _CHUNK)]], buf, sem)

    def store(c, buf, sem):
        pltpu.async_copy(buf, out_hbm.at[pl.ds(base + c * _CHUNK, _CHUNK)],
                         sem)

    for b in range(_NB):
        gather(b, bufs[b], gsems[b])

    def body(i, carry):
        for b in range(_NB):
            c = i * _NB + b
            pltpu.make_async_copy(bufs[b], out_hbm.at[pl.ds(0, _CHUNK)],
                                  gsems[b]).wait()
            store(c, bufs[b], ssems[b])
            pltpu.make_async_copy(bufs[b], out_hbm.at[pl.ds(0, _CHUNK)],
                                  ssems[b]).wait()
            nxt = c + _NB

            @pl.when(nxt < _NCHUNK)
            def _():
                gather(nxt, bufs[b], gsems[b])

        return carry

    lax.fori_loop(0, _NCHUNK // _NB, body, 0)


def kernel(indices, table):
    flat_idx = indices.reshape(_B).astype(jnp.int32)
    out = _gather_kernel(flat_idx, table)
    return out.reshape(_BATCH, _HIST, _EMBED)


# speedup vs baseline: 1.8959x; 1.0002x over previous
import functools

import jax
import jax.numpy as jnp
from jax import lax
from jax.experimental import pallas as pl
from jax.experimental.pallas import tpu as pltpu
from jax.experimental.pallas import tpu_sc as plsc

_VOCAB = 100000
_EMBED = 128
_BATCH = 4096
_HIST = 200
_B = _BATCH * _HIST
_NC = 2
_NS = 16
_NW = _NC * _NS
_B_PER_W = _B // _NW
_CHUNK = 200
_NCHUNK = _B_PER_W // _CHUNK
_NB = 4

_mesh = plsc.VectorSubcoreMesh(core_axis_name="c", subcore_axis_name="s")


@functools.partial(
    pl.kernel,
    out_type=jax.ShapeDtypeStruct((_B, _EMBED), jnp.float32),
    mesh=_mesh,
    scratch_types=[
        pltpu.VMEM((_B_PER_W,), jnp.int32),
        [pltpu.VMEM((_CHUNK, _EMBED), jnp.float32) for _ in range(_NB)],
        [pltpu.SemaphoreType.DMA for _ in range(_NB)],
        [pltpu.SemaphoreType.DMA for _ in range(_NB)],
    ],
)
def _gather_kernel(idx_hbm, table_hbm, out_hbm, idx_v, bufs, gsems, ssems):
    wid = lax.axis_index("s") * _NC + lax.axis_index("c")
    base = wid * _B_PER_W

    pltpu.sync_copy(idx_hbm.at[pl.ds(base, _B_PER_W)], idx_v)

    def gather(c, buf, sem):
        pltpu.async_copy(
            table_hbm.at[idx_v.at[pl.ds(c * _CHUNK, _CHUNK)]], buf, sem)

    def store(c, buf, sem):
        pltpu.async_copy(buf, out_hbm.at[pl.ds(base + c * _CHUNK, _CHUNK)],
                         sem)

    for b in range(_NB):
        gather(b, bufs[b], gsems[b])

    def body(i, carry):
        for b in range(_NB):
            c = i * _NB + b
            pltpu.make_async_copy(bufs[b], out_hbm.at[pl.ds(0, _CHUNK)],
                                  gsems[b]).wait()
            store(c, bufs[b], ssems[b])
            pltpu.make_async_copy(bufs[b], out_hbm.at[pl.ds(0, _CHUNK)],
                                  ssems[b]).wait()
            nxt = c + _NB

            @pl.when(nxt < _NCHUNK)
            def _():
                gather(nxt, bufs[b], gsems[b])

        return carry

    lax.fori_loop(0, _NCHUNK // _NB, body, 0)


def kernel(indices, table):
    flat_idx = indices.reshape(_B).astype(jnp.int32)
    out = _gather_kernel(flat_idx, table)
    return out.reshape(_BATCH, _HIST, _EMBED)
